# speedup vs baseline: 2.9327x; 2.9327x over previous
import functools

import jax
import jax.numpy as jnp
from jax import lax
from jax.experimental import pallas as pl
from jax.experimental.pallas import tpu as pltpu
from jax.experimental.pallas import tpu_sc as plsc

D = 128
IPG = 128
CHUNK = 256


@functools.lru_cache(maxsize=None)
def _make_gather(B: int):
    info = plsc.get_sparse_core_info()
    NC, NS = info.num_cores, info.num_subcores
    NW = NC * NS
    b_per_w = B // NW
    assert B % (NW * CHUNK) == 0
    n_chunks = b_per_w // CHUNK
    mesh = plsc.VectorSubcoreMesh(core_axis_name="c", subcore_axis_name="s")

    @functools.partial(
        pl.kernel,
        out_type=jax.ShapeDtypeStruct((B, D), jnp.float32),
        mesh=mesh,
        scratch_types=[
            pltpu.VMEM((CHUNK // IPG, IPG), jnp.int32),
            pltpu.VMEM((CHUNK, D), jnp.float32),
            pltpu.SemaphoreType.DMA,
        ],
    )
    def gather_kernel(table_hbm, idx_hbm, out_hbm, idx_v, rows_v, sem):
        wid = lax.axis_index("s") * NC + lax.axis_index("c")
        idx_row0 = wid * (b_per_w // IPG)
        out_base = wid * b_per_w

        def body(i, carry):
            pltpu.sync_copy(
                idx_hbm.at[pl.ds(idx_row0 + i * (CHUNK // IPG), CHUNK // IPG)],
                idx_v,
            )
            for j in range(CHUNK // IPG):
                pltpu.async_copy(
                    table_hbm.at[idx_v.at[j]],
                    rows_v.at[pl.ds(j * IPG, IPG)],
                    sem,
                ).wait()
            pltpu.sync_copy(rows_v, out_hbm.at[pl.ds(out_base + i * CHUNK, CHUNK)])
            return carry

        lax.fori_loop(0, n_chunks, body, 0)

    return gather_kernel


def kernel(indices, feature_tensor):
    B = indices.size
    idx2d = indices.reshape(B // IPG, IPG).astype(jnp.int32)
    out = _make_gather(B)(feature_tensor, idx2d)
    return out.reshape(indices.shape + (D,))


# speedup vs baseline: 3.4516x; 1.1770x over previous
import functools

import jax
import jax.numpy as jnp
from jax import lax
from jax.experimental import pallas as pl
from jax.experimental.pallas import tpu as pltpu
from jax.experimental.pallas import tpu_sc as plsc

D = 128
IPG = 128
K = 2
GROUP = K * IPG


@functools.lru_cache(maxsize=None)
def _make_gather(B: int):
    info = plsc.get_sparse_core_info()
    NC, NS = info.num_cores, info.num_subcores
    NW = NC * NS
    b_per_w = B // NW
    n_chunks_w = b_per_w // IPG
    G = b_per_w // GROUP
    assert B % (NW * GROUP) == 0 and G >= 4 and G % 2 == 0
    mesh = plsc.VectorSubcoreMesh(core_axis_name="c", subcore_axis_name="s")

    @functools.partial(
        pl.kernel,
        out_type=jax.ShapeDtypeStruct((B, D), jnp.float32),
        mesh=mesh,
        scratch_types=[
            pltpu.VMEM((n_chunks_w, IPG), jnp.int32),
            pltpu.VMEM((GROUP, D), jnp.float32),
            pltpu.VMEM((GROUP, D), jnp.float32),
            pltpu.SemaphoreType.DMA,
            pltpu.SemaphoreType.DMA,
            pltpu.SemaphoreType.DMA,
            pltpu.SemaphoreType.DMA,
        ],
    )
    def gather_kernel(table_hbm, idx_hbm, out_hbm,
                      idx_all, rows0, rows1, sg0, sg1, so0, so1):
        wid = lax.axis_index("s") * NC + lax.axis_index("c")
        rbase = wid * b_per_w

        pltpu.sync_copy(idx_hbm.at[pl.ds(wid * n_chunks_w, n_chunks_w)], idx_all)

        rows = (rows0, rows1)
        sg = (sg0, sg1)
        so = (so0, so1)

        def g_desc(g, bank, h):
            return pltpu.make_async_copy(
                table_hbm.at[idx_all.at[g * K + h]],
                rows[bank].at[pl.ds(h * IPG, IPG)],
                sg[bank],
            )

        def s_desc(g, bank):
            return pltpu.make_async_copy(
                rows[bank],
                out_hbm.at[pl.ds(rbase + g * GROUP, GROUP)],
                so[bank],
            )

        def gath(g, bank):
            for h in range(K):
                g_desc(g, bank, h).start()

        def gath_wait(g, bank):
            for h in range(K):
                g_desc(g, bank, h).wait()

        def stage(g, skip_b1_store_wait=False, last=False):
            gath_wait(g, 0)
            s_desc(g, 0).start()
            if not skip_b1_store_wait:
                s_desc(g - 1, 1).wait()
            gath(g + 1, 1)
            gath_wait(g + 1, 1)
            s_desc(g + 1, 1).start()
            s_desc(g, 0).wait()
            if not last:
                gath(g + 2, 0)
            else:
                s_desc(g + 1, 1).wait()

        gath(0, 0)
        stage(0, skip_b1_store_wait=True)

        def body(u, carry):
            stage(2 * u)
            return carry

        lax.fori_loop(1, G // 2 - 1, body, 0)
        stage(G - 2, last=True)

    return gather_kernel


def kernel(indices, feature_tensor):
    B = indices.size
    idx2d = indices.reshape(B // IPG, IPG).astype(jnp.int32)
    out = _make_gather(B)(feature_tensor, idx2d)
    return out.reshape(indices.shape + (D,))


# speedup vs baseline: 11.8874x; 3.4440x over previous
import functools

import jax
import jax.numpy as jnp
from jax import lax
from jax.experimental import pallas as pl
from jax.experimental.pallas import tpu as pltpu
from jax.experimental.pallas import tpu_sc as plsc

D = 128
IPG = 128
K = 2
GROUP = K * IPG


@functools.lru_cache(maxsize=None)
def _make_gather(B: int):
    info = plsc.get_sparse_core_info()
    NC, NS = info.num_cores, info.num_subcores
    NW = NC * NS
    b_per_w = B // NW
    n_chunks_w = b_per_w // IPG
    G = b_per_w // GROUP
    assert B % (NW * GROUP) == 0 and G >= 4 and G % 2 == 0
    mesh = plsc.VectorSubcoreMesh(core_axis_name="c", subcore_axis_name="s")

    @functools.partial(
        pl.kernel,
        out_type=jax.ShapeDtypeStruct((B, D), jnp.float32),
        mesh=mesh,
        scratch_types=[
            pltpu.VMEM((n_chunks_w, IPG), jnp.int32),
            pltpu.VMEM((GROUP, D), jnp.float32),
            pltpu.VMEM((GROUP, D), jnp.float32),
            pltpu.SemaphoreType.DMA,
            pltpu.SemaphoreType.DMA,
            pltpu.SemaphoreType.DMA,
            pltpu.SemaphoreType.DMA,
        ],
    )
    def gather_kernel(table_hbm, idx_hbm, out_hbm,
                      idx_all, rows0, rows1, sg0, sg1, so0, so1):
        wid = lax.axis_index("s") * NC + lax.axis_index("c")
        rbase = wid * b_per_w

        pltpu.sync_copy(idx_hbm.at[pl.ds(wid * n_chunks_w, n_chunks_w)], idx_all)

        rows = (rows0, rows1)
        sg = (sg0, sg1)
        so = (so0, so1)

        def g_desc(g, bank, h):
            return pltpu.make_async_copy(
                table_hbm.at[idx_all.at[g * K + h]],
                rows[bank].at[pl.ds(h * IPG, IPG)],
                sg[bank],
            )

        def s_desc(g, bank):
            return pltpu.make_async_copy(
                rows[bank],
                out_hbm.at[pl.ds(rbase + g * GROUP, GROUP)],
                so[bank],
            )

        def gath(g, bank):
            for h in range(K):
                g_desc(g, bank, h).start()

        def gath_wait(g, bank):
            for h in range(K):
                g_desc(g, bank, h).wait()

        def stage(g, skip_b1_store_wait=False, last=False):
            gath_wait(g, 0)
            s_desc(g, 0).start()
            if not skip_b1_store_wait:
                s_desc(g - 1, 1).wait()
            gath(g + 1, 1)
            gath_wait(g + 1, 1)
            s_desc(g + 1, 1).start()
            s_desc(g, 0).wait()
            if not last:
                gath(g + 2, 0)
            else:
                s_desc(g + 1, 1).wait()

        gath(0, 0)
        stage(0, skip_b1_store_wait=True)

        def body(u, carry):
            stage(2 * u)
            return carry

        lax.fori_loop(1, G // 2 - 1, body, 0)
        stage(G - 2, last=True)

    return gather_kernel


def kernel(indices, feature_tensor):
    B = indices.size
    nb, nt = indices.shape
    idx2d = indices.T.reshape(B // IPG, IPG).astype(jnp.int32)
    out = _make_gather(B)(feature_tensor, idx2d)
    return out.reshape(nt, nb, D).transpose(1, 0, 2)


# speedup vs baseline: 11.9024x; 1.0013x over previous
import functools

import jax
import jax.numpy as jnp
from jax import lax
from jax.experimental import pallas as pl
from jax.experimental.pallas import tpu as pltpu
from jax.experimental.pallas import tpu_sc as plsc

D = 128
IPG = 128
K = 2
GROUP = K * IPG
NBANK = 3


@functools.lru_cache(maxsize=None)
def _make_gather(B: int):
    info = plsc.get_sparse_core_info()
    NC, NS = info.num_cores, info.num_subcores
    NW = NC * NS
    b_per_w = B // NW
    n_chunks_w = b_per_w // IPG
    G = b_per_w // GROUP
    assert B % (NW * GROUP) == 0 and G >= 6 and (G - 4) % NBANK == 0
    mesh = plsc.VectorSubcoreMesh(core_axis_name="c", subcore_axis_name="s")

    @functools.partial(
        pl.kernel,
        out_type=jax.ShapeDtypeStruct((B, D), jnp.float32),
        mesh=mesh,
        scratch_types=[
            pltpu.VMEM((n_chunks_w, IPG), jnp.int32),
            pltpu.VMEM((GROUP, D), jnp.float32),
            pltpu.VMEM((GROUP, D), jnp.float32),
            pltpu.VMEM((GROUP, D), jnp.float32),
            pltpu.SemaphoreType.DMA,
            pltpu.SemaphoreType.DMA,
            pltpu.SemaphoreType.DMA,
            pltpu.SemaphoreType.DMA,
            pltpu.SemaphoreType.DMA,
            pltpu.SemaphoreType.DMA,
        ],
    )
    def gather_kernel(table_hbm, idx_hbm, out_hbm,
                      idx_all, rows0, rows1, rows2,
                      sg0, sg1, sg2, so0, so1, so2):
        wid = lax.axis_index("s") * NC + lax.axis_index("c")
        rbase = wid * b_per_w

        pltpu.sync_copy(idx_hbm.at[pl.ds(wid * n_chunks_w, n_chunks_w)], idx_all)

        rows = (rows0, rows1, rows2)
        sg = (sg0, sg1, sg2)
        so = (so0, so1, so2)

        def g_desc(g, bank, h):
            return pltpu.make_async_copy(
                table_hbm.at[idx_all.at[g * K + h]],
                rows[bank].at[pl.ds(h * IPG, IPG)],
                sg[bank],
            )

        def s_desc(g, bank):
            return pltpu.make_async_copy(
                rows[bank],
                out_hbm.at[pl.ds(rbase + g * GROUP, GROUP)],
                so[bank],
            )

        def gath(g, bank):
            for h in range(K):
                g_desc(g, bank, h).start()

        def gath_wait(g, bank):
            for h in range(K):
                g_desc(g, bank, h).wait()

        def step(g, bank, first=False, fire=True):
            gath_wait(g, bank)
            s_desc(g, bank).start()
            if not first:
                s_desc(g - 1, (bank - 1) % NBANK).wait()
            if fire:
                gath(g + 2, (bank + 2) % NBANK)

        gath(0, 0)
        gath(1, 1)
        step(0, 0, first=True)

        def body(u, carry):
            g = 1 + NBANK * u
            step(g + 0, 1)
            step(g + 1, 2)
            step(g + 2, 0)
            return carry

        lax.fori_loop(0, (G - 4) // NBANK, body, 0)

        b3 = (G - 3) % NBANK
        step(G - 3, b3, fire=True)
        step(G - 2, (b3 + 1) % NBANK, fire=False)
        step(G - 1, (b3 + 2) % NBANK, fire=False)
        s_desc(G - 1, (b3 + 2) % NBANK).wait()

    return gather_kernel


def kernel(indices, feature_tensor):
    B = indices.size
    nb, nt = indices.shape
    idx2d = indices.T.reshape(B // IPG, IPG).astype(jnp.int32)
    out = _make_gather(B)(feature_tensor, idx2d)
    return out.reshape(nt, nb, D).transpose(1, 0, 2)
